# Initial kernel scaffold; baseline (speedup 1.0000x reference)
import jax
import jax.numpy as jnp
from jax.experimental import pallas as pl


def kernel(query, reference_points, input_flatten, input_spatial_shapes, input_level_start_index, W_off, b_off, W_att, b_att, W_val, b_val, W_out, b_out):
    raise NotImplementedError("write your pallas kernel here")


# speedup vs baseline: 133.2187x; 133.2187x over previous
import functools

import jax
import jax.numpy as jnp
from jax import lax
from jax.experimental import pallas as pl
from jax.experimental.pallas import tpu as pltpu
from jax.experimental.pallas import tpu_sc as plsc

D_MODEL = 256
N_HEADS = 8
N_LEVELS = 4
N_POINTS = 4
D_HEAD = 32
HALF = 16
SHAPES = ((64, 64), (32, 32), (16, 16), (8, 8))
STARTS = (0, 4096, 5120, 5376)
LIN = 5440
T = 4
LINP = 5632
QB = 512
NQB = LINP // QB
SC_QB = 128
SC_NB = LINP // SC_QB
SC_G = SC_QB // 16


def _vsum_body(x_ref, wv_ref, bv_ref, out_ref):
    x = x_ref[...]
    w = wv_ref[...]
    b = bv_ref[...]
    x0, x1, x2, x3 = x[0], x[1], x[2], x[3]
    sums = (x0 + x1, x0 + x1 + x2, x1 + x2 + x3, x2 + x3)
    k = (2.0, 3.0, 3.0, 2.0)
    outs = [lax.dot_general(w, sums[t], (((0,), (1,)), ((), ())),
                            preferred_element_type=jnp.float32) + k[t] * b
            for t in range(T)]
    out_ref[...] = jnp.stack(outs, axis=0)


_vsum_call = pl.pallas_call(
    _vsum_body,
    grid=(NQB,),
    in_specs=[pl.BlockSpec((T, QB, D_MODEL), lambda i: (0, i, 0)),
              pl.BlockSpec((D_MODEL, D_MODEL), lambda i: (0, 0)),
              pl.BlockSpec((D_MODEL, 1), lambda i: (0, 0))],
    out_specs=pl.BlockSpec((T, D_MODEL, QB), lambda i: (0, 0, i)),
    out_shape=jax.ShapeDtypeStruct((T, D_MODEL, LINP), jnp.float32),
)


def _prep_body(q_ref, rp_ref, watt_ref, batt_ref, woff_ref, boff_ref,
               idx_ref, w_ref):
    t1 = pl.program_id(0)
    q = q_ref[0]
    logits = lax.dot_general(watt_ref[...], q, (((0,), (1,)), ((), ())))
    logits = logits + batt_ref[...]
    a = logits.reshape(N_HEADS, N_LEVELS * N_POINTS, QB)
    m = jnp.max(a, axis=1, keepdims=True)
    e = jnp.exp(a - m)
    s = jnp.sum(e, axis=1, keepdims=True)
    kinv = jnp.where((t1 == 0) | (t1 == 3), 0.5, 1.0 / 3.0)
    aw = (e / s * kinv).reshape(N_HEADS, N_LEVELS, N_POINTS, QB)
    offs = lax.dot_general(woff_ref[...], q, (((0,), (1,)), ((), ())))
    offs = (offs + boff_ref[...]).reshape(N_HEADS, N_LEVELS, N_POINTS, 2, QB)
    rp = rp_ref[0]

    idx_c = [[], [], [], []]
    w_c = [[], [], [], []]
    for l in range(N_LEVELS):
        hl, wl = SHAPES[l]
        st = STARTS[l]
        px = rp[l, 0][None, None] * wl + offs[:, l, :, 0, :] - 0.5
        py = rp[l, 1][None, None] * hl + offs[:, l, :, 1, :] - 0.5
        x0 = jnp.floor(px)
        y0 = jnp.floor(py)
        fx = px - x0
        fy = py - y0
        vx0 = ((x0 >= 0) & (x0 <= wl - 1)).astype(jnp.float32)
        vx1 = ((x0 >= -1) & (x0 <= wl - 2)).astype(jnp.float32)
        vy0 = ((y0 >= 0) & (y0 <= hl - 1)).astype(jnp.float32)
        vy1 = ((y0 >= -1) & (y0 <= hl - 2)).astype(jnp.float32)
        xi0 = jnp.clip(x0, 0, wl - 1).astype(jnp.int32)
        xi1 = jnp.clip(x0 + 1, 0, wl - 1).astype(jnp.int32)
        yi0 = jnp.clip(y0, 0, hl - 1).astype(jnp.int32)
        yi1 = jnp.clip(y0 + 1, 0, hl - 1).astype(jnp.int32)
        awl = aw[:, l]
        rows = (st + yi0 * wl + xi0, st + yi0 * wl + xi1,
                st + yi1 * wl + xi0, st + yi1 * wl + xi1)
        wts = (awl * (1 - fx) * (1 - fy) * vx0 * vy0,
               awl * fx * (1 - fy) * vx1 * vy0,
               awl * (1 - fx) * fy * vx0 * vy1,
               awl * fx * fy * vx1 * vy1)
        for c in range(4):
            idx_c[c].append(rows[c])
            w_c[c].append(wts[c])

    idx_all = jnp.stack([jnp.stack(v, axis=1) for v in idx_c], axis=0)
    w_all = jnp.stack([jnp.stack(v, axis=1) for v in w_c], axis=0)
    idx_ref[...] = idx_all[None]
    w_ref[...] = w_all[None]


_prep_call = pl.pallas_call(
    _prep_body,
    grid=(T, NQB),
    in_specs=[pl.BlockSpec((1, QB, D_MODEL), lambda t, i: (t, i, 0)),
              pl.BlockSpec((1, N_LEVELS, 2, QB), lambda t, i: (t, 0, 0, i)),
              pl.BlockSpec((D_MODEL, 128), lambda t, i: (0, 0)),
              pl.BlockSpec((128, 1), lambda t, i: (0, 0)),
              pl.BlockSpec((D_MODEL, D_MODEL), lambda t, i: (0, 0)),
              pl.BlockSpec((D_MODEL, 1), lambda t, i: (0, 0))],
    out_specs=[pl.BlockSpec((1, 4, N_HEADS, N_LEVELS, N_POINTS, QB),
                            lambda t, i: (t, 0, 0, 0, 0, i)),
               pl.BlockSpec((1, 4, N_HEADS, N_LEVELS, N_POINTS, QB),
                            lambda t, i: (t, 0, 0, 0, 0, i))],
    out_shape=[jax.ShapeDtypeStruct((T, 4, N_HEADS, N_LEVELS, N_POINTS, LINP),
                                    jnp.int32),
               jax.ShapeDtypeStruct((T, 4, N_HEADS, N_LEVELS, N_POINTS, LINP),
                                    jnp.float32)],
)


_SC_NC = 2
_SC_NS = 16


def _sc_body(vsum_hbm, idx_hbm, w_hbm, out_hbm, map_v, idx_v, w_v, out_v):
    wid = lax.axis_index("s") * _SC_NC + lax.axis_index("c")
    for r in range(2):
        combo = wid * 2 + r
        t1 = combo // 16
        rem = combo % 16
        h = rem // 2
        half = rem % 2
        c0 = pl.multiple_of(h * 32 + half * 16, HALF)
        pltpu.sync_copy(vsum_hbm.at[t1, pl.ds(c0, HALF), :], map_v)

        def blk_body(blk, carry):
            q0 = pl.multiple_of(blk * SC_QB, SC_QB)
            for c in range(4):
                pltpu.sync_copy(idx_hbm.at[t1, c, h, :, :, pl.ds(q0, SC_QB)],
                                idx_v.at[c])
                pltpu.sync_copy(w_hbm.at[t1, c, h, :, :, pl.ds(q0, SC_QB)],
                                w_v.at[c])

            def g_body(g, carry2):
                acc = [jnp.zeros((16,), jnp.float32) for _ in range(HALF)]
                for l in range(N_LEVELS):
                    for p in range(N_POINTS):
                        for c in range(4):
                            rows = idx_v[c, l, p, pl.ds(g * 16, 16)]
                            wv = w_v[c, l, p, pl.ds(g * 16, 16)]
                            for ch in range(HALF):
                                col = jnp.full((16,), ch, jnp.int32)
                                gat = plsc.load_gather(map_v, [col, rows])
                                acc[ch] = acc[ch] + wv * gat
                for ch in range(HALF):
                    out_v[ch, pl.ds(g * 16, 16)] = acc[ch]
                return carry2

            lax.fori_loop(0, SC_G, g_body, 0, unroll=False)
            pltpu.sync_copy(out_v, out_hbm.at[t1, h, half, :, pl.ds(q0, SC_QB)])
            return carry

        lax.fori_loop(0, SC_NB, blk_body, 0, unroll=False)


_sc_call = functools.partial(
    pl.kernel,
    out_type=jax.ShapeDtypeStruct((T, N_HEADS, 2, HALF, LINP), jnp.float32),
    mesh=plsc.VectorSubcoreMesh(core_axis_name="c", subcore_axis_name="s"),
    compiler_params=pltpu.CompilerParams(use_tc_tiling_on_sc=False,
                                         needs_layout_passes=False),
    scratch_types=[pltpu.VMEM((HALF, LINP), jnp.float32),
                   pltpu.VMEM((4, N_LEVELS, N_POINTS, SC_QB), jnp.int32),
                   pltpu.VMEM((4, N_LEVELS, N_POINTS, SC_QB), jnp.float32),
                   pltpu.VMEM((HALF, SC_QB), jnp.float32)],
)(_sc_body)


def _out_body(x_ref, wout_ref, bout_ref, out_ref):
    x = x_ref[0]
    y = lax.dot_general(x, wout_ref[...], (((0,), (0,)), ((), ())),
                        preferred_element_type=jnp.float32)
    out_ref[...] = (y + bout_ref[...])[None]


_out_call = pl.pallas_call(
    _out_body,
    grid=(T, NQB),
    in_specs=[pl.BlockSpec((1, D_MODEL, QB), lambda t, i: (t, 0, i)),
              pl.BlockSpec((D_MODEL, D_MODEL), lambda t, i: (0, 0)),
              pl.BlockSpec((1, D_MODEL), lambda t, i: (0, 0))],
    out_specs=pl.BlockSpec((1, QB, D_MODEL), lambda t, i: (t, i, 0)),
    out_shape=jax.ShapeDtypeStruct((T, LINP, D_MODEL), jnp.float32),
)


def kernel(query, reference_points, input_flatten, input_spatial_shapes,
           input_level_start_index, W_off, b_off, W_att, b_att, W_val, b_val,
           W_out, b_out):
    xf = jnp.pad(input_flatten[0], ((0, 0), (0, LINP - LIN), (0, 0)))
    q4 = jnp.pad(query[0], ((0, 0), (0, LINP - LIN), (0, 0)))
    rpt = jnp.transpose(reference_points[0], (0, 2, 3, 1))
    rpt = jnp.pad(rpt, ((0, 0), (0, 0), (0, 0), (0, LINP - LIN)))

    vsum = _vsum_call(xf, W_val, b_val.reshape(D_MODEL, 1))
    idx_all, w_all = _prep_call(q4, rpt, W_att, b_att.reshape(128, 1),
                                W_off, b_off.reshape(D_MODEL, 1))
    sc_out = _sc_call(vsum, idx_all, w_all)
    x_t = sc_out.reshape(T, D_MODEL, LINP)
    out = _out_call(x_t, W_out, b_out.reshape(1, D_MODEL))
    return out[:, :LIN, :].reshape(1, T, LIN, D_MODEL)


# speedup vs baseline: 264.4470x; 1.9851x over previous
import functools

import jax
import jax.numpy as jnp
from jax import lax
from jax.experimental import pallas as pl
from jax.experimental.pallas import tpu as pltpu
from jax.experimental.pallas import tpu_sc as plsc

D_MODEL = 256
N_HEADS = 8
N_LEVELS = 4
N_POINTS = 4
D_HEAD = 32
HALF = 16
SHAPES = ((64, 64), (32, 32), (16, 16), (8, 8))
STARTS = (0, 4096, 5120, 5376)
LIN = 5440
T = 4
LINP = 5632
QB = 512
NQB = LINP // QB
SC_QB = 64
SC_NB = LINP // SC_QB
SC_G = SC_QB // 16


def _vsum_body(x_ref, wv_ref, bv_ref, out_ref):
    x = x_ref[...]
    w = wv_ref[...]
    b = bv_ref[...]
    x0, x1, x2, x3 = x[0], x[1], x[2], x[3]
    sums = (x0 + x1, x0 + x1 + x2, x1 + x2 + x3, x2 + x3)
    k = (2.0, 3.0, 3.0, 2.0)
    outs = [lax.dot_general(w, sums[t], (((0,), (1,)), ((), ())),
                            preferred_element_type=jnp.float32) + k[t] * b
            for t in range(T)]
    out_ref[...] = jnp.stack(outs, axis=0)


_vsum_call = pl.pallas_call(
    _vsum_body,
    grid=(NQB,),
    in_specs=[pl.BlockSpec((T, QB, D_MODEL), lambda i: (0, i, 0)),
              pl.BlockSpec((D_MODEL, D_MODEL), lambda i: (0, 0)),
              pl.BlockSpec((D_MODEL, 1), lambda i: (0, 0))],
    out_specs=pl.BlockSpec((T, D_MODEL, QB), lambda i: (0, 0, i)),
    out_shape=jax.ShapeDtypeStruct((T, D_MODEL, LINP), jnp.float32),
)


def _prep_body(q_ref, rp_ref, watt_ref, batt_ref, woff_ref, boff_ref,
               idx_ref, w_ref):
    t1 = pl.program_id(0)
    q = q_ref[0]
    logits = lax.dot_general(watt_ref[...], q, (((0,), (1,)), ((), ())))
    logits = logits + batt_ref[...]
    a = logits.reshape(N_HEADS, N_LEVELS * N_POINTS, QB)
    m = jnp.max(a, axis=1, keepdims=True)
    e = jnp.exp(a - m)
    s = jnp.sum(e, axis=1, keepdims=True)
    kinv = jnp.where((t1 == 0) | (t1 == 3), 0.5, 1.0 / 3.0)
    aw = (e / s * kinv).reshape(N_HEADS, N_LEVELS, N_POINTS, QB)
    offs = lax.dot_general(woff_ref[...], q, (((0,), (1,)), ((), ())))
    offs = (offs + boff_ref[...]).reshape(N_HEADS, N_LEVELS, N_POINTS, 2, QB)
    rp = rp_ref[0]

    idx_c = [[], [], [], []]
    w_c = [[], [], [], []]
    for l in range(N_LEVELS):
        hl, wl = SHAPES[l]
        st = STARTS[l]
        px = rp[l, 0][None, None] * wl + offs[:, l, :, 0, :] - 0.5
        py = rp[l, 1][None, None] * hl + offs[:, l, :, 1, :] - 0.5
        x0 = jnp.floor(px)
        y0 = jnp.floor(py)
        fx = px - x0
        fy = py - y0
        vx0 = ((x0 >= 0) & (x0 <= wl - 1)).astype(jnp.float32)
        vx1 = ((x0 >= -1) & (x0 <= wl - 2)).astype(jnp.float32)
        vy0 = ((y0 >= 0) & (y0 <= hl - 1)).astype(jnp.float32)
        vy1 = ((y0 >= -1) & (y0 <= hl - 2)).astype(jnp.float32)
        xi0 = jnp.clip(x0, 0, wl - 1).astype(jnp.int32)
        xi1 = jnp.clip(x0 + 1, 0, wl - 1).astype(jnp.int32)
        yi0 = jnp.clip(y0, 0, hl - 1).astype(jnp.int32)
        yi1 = jnp.clip(y0 + 1, 0, hl - 1).astype(jnp.int32)
        awl = aw[:, l]
        rows = (st + yi0 * wl + xi0, st + yi0 * wl + xi1,
                st + yi1 * wl + xi0, st + yi1 * wl + xi1)
        wts = (awl * (1 - fx) * (1 - fy) * vx0 * vy0,
               awl * fx * (1 - fy) * vx1 * vy0,
               awl * (1 - fx) * fy * vx0 * vy1,
               awl * fx * fy * vx1 * vy1)
        for c in range(4):
            idx_c[c].append(rows[c])
            w_c[c].append(wts[c])

    idx_all = jnp.stack([jnp.stack(v, axis=1) for v in idx_c], axis=0)
    w_all = jnp.stack([jnp.stack(v, axis=1) for v in w_c], axis=0)
    idx_ref[...] = idx_all[None]
    w_ref[...] = w_all[None]


_prep_call = pl.pallas_call(
    _prep_body,
    grid=(T, NQB),
    in_specs=[pl.BlockSpec((1, QB, D_MODEL), lambda t, i: (t, i, 0)),
              pl.BlockSpec((1, N_LEVELS, 2, QB), lambda t, i: (t, 0, 0, i)),
              pl.BlockSpec((D_MODEL, 128), lambda t, i: (0, 0)),
              pl.BlockSpec((128, 1), lambda t, i: (0, 0)),
              pl.BlockSpec((D_MODEL, D_MODEL), lambda t, i: (0, 0)),
              pl.BlockSpec((D_MODEL, 1), lambda t, i: (0, 0))],
    out_specs=[pl.BlockSpec((1, 4, N_HEADS, N_LEVELS, N_POINTS, QB),
                            lambda t, i: (t, 0, 0, 0, 0, i)),
               pl.BlockSpec((1, 4, N_HEADS, N_LEVELS, N_POINTS, QB),
                            lambda t, i: (t, 0, 0, 0, 0, i))],
    out_shape=[jax.ShapeDtypeStruct((T, 4, N_HEADS, N_LEVELS, N_POINTS, LINP),
                                    jnp.int32),
               jax.ShapeDtypeStruct((T, 4, N_HEADS, N_LEVELS, N_POINTS, LINP),
                                    jnp.float32)],
)


_SC_NC = 2
_SC_NS = 16


def _sc_body(vsum_hbm, idx_hbm, w_hbm, out_hbm, map_v, idx_v, w_v, out_v,
             insem0, insem1, outsem0, outsem1):
    wid = lax.axis_index("s") * _SC_NC + lax.axis_index("c")
    insems = (insem0, insem1)
    outsems = (outsem0, outsem1)
    for r in range(2):
        combo = wid * 2 + r
        t1 = combo // 16
        rem = combo % 16
        h = rem // 2
        half = rem % 2
        c0 = pl.multiple_of(h * 32 + half * 16, HALF)
        pltpu.sync_copy(vsum_hbm.at[t1, pl.ds(c0, HALF), pl.ds(0, LIN)], map_v)

        def in_copies(blk, buf):
            q0 = pl.multiple_of(blk * SC_QB, SC_QB)
            for c in range(4):
                yield (idx_hbm.at[t1, c, h, :, :, pl.ds(q0, SC_QB)],
                       idx_v.at[buf, c], insems[buf])
                yield (w_hbm.at[t1, c, h, :, :, pl.ds(q0, SC_QB)],
                       w_v.at[buf, c], insems[buf])

        def issue_in(blk, buf):
            for s, dx, sem in in_copies(blk, buf):
                pltpu.async_copy(s, dx, sem)

        def wait_in(blk, buf):
            for s, dx, sem in in_copies(blk, buf):
                pltpu.make_async_copy(s, dx, sem).wait()

        def out_copy(blk, buf):
            q0 = pl.multiple_of(blk * SC_QB, SC_QB)
            return (out_v.at[buf],
                    out_hbm.at[t1, h, half, :, pl.ds(q0, SC_QB)], outsems[buf])

        def compute(blk, buf):
            for g in range(SC_G):
                def j_body(j, accs):
                    l = j // N_POINTS
                    p = j % N_POINTS
                    new = list(accs)
                    for c in range(4):
                        rows = idx_v[buf, c, l, p, pl.ds(g * 16, 16)]
                        wv = w_v[buf, c, l, p, pl.ds(g * 16, 16)]
                        for ch in range(HALF):
                            col = jnp.full((16,), ch, jnp.int32)
                            gat = plsc.load_gather(map_v, [col, rows])
                            new[ch] = new[ch] + wv * gat
                    return tuple(new)

                accs = lax.fori_loop(
                    0, N_LEVELS * N_POINTS, j_body,
                    tuple(jnp.zeros((16,), jnp.float32) for _ in range(HALF)))
                for ch in range(HALF):
                    out_v[buf, ch, pl.ds(g * 16, 16)] = accs[ch]

        issue_in(0, 0)

        def pair_body(p_i, carry):
            for sub in range(2):
                blk = p_i * 2 + sub
                wait_in(blk, sub)

                @pl.when(blk + 1 < SC_NB)
                def _():
                    issue_in(blk + 1, 1 - sub)

                @pl.when(p_i > 0)
                def _():
                    s, dx, sem = out_copy(blk, sub)
                    pltpu.make_async_copy(s, dx, sem).wait()

                compute(blk, sub)
                s, dx, sem = out_copy(blk, sub)
                pltpu.async_copy(s, dx, sem)
            return carry

        lax.fori_loop(0, SC_NB // 2, pair_body, 0, unroll=False)
        for sub in range(2):
            s, dx, sem = out_copy(SC_NB - 2 + sub, sub)
            pltpu.make_async_copy(s, dx, sem).wait()


_sc_call = functools.partial(
    pl.kernel,
    out_type=jax.ShapeDtypeStruct((T, N_HEADS, 2, HALF, LINP), jnp.float32),
    mesh=plsc.VectorSubcoreMesh(core_axis_name="c", subcore_axis_name="s"),
    compiler_params=pltpu.CompilerParams(use_tc_tiling_on_sc=False,
                                         needs_layout_passes=False),
    scratch_types=[pltpu.VMEM((HALF, LIN), jnp.float32),
                   pltpu.VMEM((2, 4, N_LEVELS, N_POINTS, SC_QB), jnp.int32),
                   pltpu.VMEM((2, 4, N_LEVELS, N_POINTS, SC_QB), jnp.float32),
                   pltpu.VMEM((2, HALF, SC_QB), jnp.float32),
                   pltpu.SemaphoreType.DMA,
                   pltpu.SemaphoreType.DMA,
                   pltpu.SemaphoreType.DMA,
                   pltpu.SemaphoreType.DMA],
)(_sc_body)


def _out_body(x_ref, wout_ref, bout_ref, out_ref):
    x = x_ref[0]
    y = lax.dot_general(x, wout_ref[...], (((0,), (0,)), ((), ())),
                        preferred_element_type=jnp.float32)
    out_ref[...] = (y + bout_ref[...])[None]


_out_call = pl.pallas_call(
    _out_body,
    grid=(T, NQB),
    in_specs=[pl.BlockSpec((1, D_MODEL, QB), lambda t, i: (t, 0, i)),
              pl.BlockSpec((D_MODEL, D_MODEL), lambda t, i: (0, 0)),
              pl.BlockSpec((1, D_MODEL), lambda t, i: (0, 0))],
    out_specs=pl.BlockSpec((1, QB, D_MODEL), lambda t, i: (t, i, 0)),
    out_shape=jax.ShapeDtypeStruct((T, LINP, D_MODEL), jnp.float32),
)


def kernel(query, reference_points, input_flatten, input_spatial_shapes,
           input_level_start_index, W_off, b_off, W_att, b_att, W_val, b_val,
           W_out, b_out):
    xf = jnp.pad(input_flatten[0], ((0, 0), (0, LINP - LIN), (0, 0)))
    q4 = jnp.pad(query[0], ((0, 0), (0, LINP - LIN), (0, 0)))
    rpt = jnp.transpose(reference_points[0], (0, 2, 3, 1))
    rpt = jnp.pad(rpt, ((0, 0), (0, 0), (0, 0), (0, LINP - LIN)))

    vsum = _vsum_call(xf, W_val, b_val.reshape(D_MODEL, 1))
    idx_all, w_all = _prep_call(q4, rpt, W_att, b_att.reshape(128, 1),
                                W_off, b_off.reshape(D_MODEL, 1))
    sc_out = _sc_call(vsum, idx_all, w_all)
    x_t = sc_out.reshape(T, D_MODEL, LINP)
    out = _out_call(x_t, W_out, b_out.reshape(1, D_MODEL))
    return out[:, :LIN, :].reshape(1, T, LIN, D_MODEL)
